# baseline (speedup 1.0000x reference)
import functools

import jax
import jax.numpy as jnp
from jax import lax
from jax.experimental import pallas as pl
from jax.experimental.pallas import tpu as pltpu
from jax.experimental.pallas import tpu_sc as plsc

VOCAB = 100000
DIM = 64
BATCH = 4096
HIST = 50

NC, NS = 2, 16
NW = NC * NS
SEQ_PW = BATCH // NW
SEQ_PC = 8
CHUNK = SEQ_PC * HIST
STEPS = SEQ_PW // SEQ_PC
SCALE = 8.0
HP = 56
DP = 2 * DIM

_mesh = plsc.VectorSubcoreMesh(core_axis_name="c", subcore_axis_name="s")


@functools.partial(
    pl.kernel,
    out_type=jax.ShapeDtypeStruct((BATCH * HP, DP), jnp.float32),
    mesh=_mesh,
    scratch_types=[
        pltpu.VMEM((SEQ_PC, DP), jnp.int32),
        pltpu.VMEM((SEQ_PC * HP + 16,), jnp.int32),
        pltpu.VMEM((SEQ_PC * HP + 16,), jnp.int32),
        pltpu.VMEM((SEQ_PC * HP, DP), jnp.float32),
        pltpu.VMEM((SEQ_PC * HP, DP), jnp.float32),
        pltpu.SemaphoreType.DMA,
        pltpu.SemaphoreType.DMA,
        pltpu.SemaphoreType.DMA,
        pltpu.SemaphoreType.DMA,
    ],
)
def _emb_lookup(x_hbm, table_hbm, out_hbm, xbuf, idx0, idx1, rows0, rows1,
                gs0, gs1, os0, os1):
    wid = lax.axis_index("s") * NC + lax.axis_index("c")
    seq_base = wid * SEQ_PW
    idx = (idx0, idx1)
    rows = (rows0, rows1)
    gsem = (gs0, gs1)
    osem = (os0, os1)

    def start_gathers(s):
        b = s % 2
        seq0 = seq_base + s * SEQ_PC
        pltpu.sync_copy(x_hbm.at[pl.ds(seq0, SEQ_PC)], xbuf)
        zeros = jnp.zeros((16,), jnp.int32)
        for i in range(SEQ_PC):
            idx[b][pl.ds(i * HP + 40, 16)] = zeros
            for col in (0, 16, 32, HIST - 16):
                idx[b][pl.ds(i * HP + col, 16)] = xbuf[i, pl.ds(col, 16)]
        return pltpu.async_copy(
            table_hbm.at[idx[b].at[pl.ds(0, SEQ_PC * HP)]],
            rows[b], gsem[b])

    gathers = [None] * STEPS
    writes = [None] * STEPS
    gathers[0] = start_gathers(0)
    for s in range(STEPS):
        b = s % 2
        if s + 1 < STEPS:
            if s >= 1:
                writes[s - 1].wait()
            gathers[s + 1] = start_gathers(s + 1)
        gathers[s].wait()

        def row(h, c):
            for i in range(SEQ_PC):
                for k in range(DIM // 16):
                    sl = pl.ds(k * 16, 16)
                    rows[b][i * HP + h, sl] = rows[b][i * HP + h, sl] * SCALE
            return c

        lax.fori_loop(0, HIST, row, 0)
        out0 = (seq_base + s * SEQ_PC) * HP
        writes[s] = pltpu.async_copy(
            rows[b], out_hbm.at[pl.ds(out0, SEQ_PC * HP)], osem[b])
    writes[STEPS - 2].wait()
    writes[STEPS - 1].wait()


def kernel(x, table):
    tbl128 = jnp.pad(table, ((0, 0), (0, DIM)))
    xp = jnp.pad(x, ((0, 0), (0, DP - HIST)))
    out = _emb_lookup(xp, tbl128)
    return out.reshape(BATCH, HP, DP)[:, :HIST, :DIM]


# speedup vs baseline: 1.0001x; 1.0001x over previous
import functools

import jax
import jax.numpy as jnp
from jax import lax
from jax.experimental import pallas as pl
from jax.experimental.pallas import tpu as pltpu
from jax.experimental.pallas import tpu_sc as plsc

VOCAB = 100000
DIM = 64
BATCH = 4096
HIST = 50

NC, NS = 2, 16
NW = NC * NS
SEQ_PW = BATCH // NW
SEQ_PC = 8
CHUNK = SEQ_PC * HIST
STEPS = SEQ_PW // SEQ_PC
SCALE = 8.0
HP = 56
DP = 2 * DIM

_mesh = plsc.VectorSubcoreMesh(core_axis_name="c", subcore_axis_name="s")


@functools.partial(
    pl.kernel,
    out_type=jax.ShapeDtypeStruct((BATCH * HP, DP), jnp.float32),
    mesh=_mesh,
    scratch_types=[
        pltpu.VMEM((SEQ_PC, DP), jnp.int32),
        pltpu.VMEM((SEQ_PC * HP + 16,), jnp.int32),
        pltpu.VMEM((SEQ_PC * HP + 16,), jnp.int32),
        pltpu.VMEM((SEQ_PC * HP, DP), jnp.float32),
        pltpu.VMEM((SEQ_PC * HP, DP), jnp.float32),
        pltpu.SemaphoreType.DMA,
        pltpu.SemaphoreType.DMA,
        pltpu.SemaphoreType.DMA,
        pltpu.SemaphoreType.DMA,
    ],
)
def _emb_lookup(x_hbm, table_hbm, out_hbm, xbuf, idx0, idx1, rows0, rows1,
                gs0, gs1, os0, os1):
    wid = lax.axis_index("s") * NC + lax.axis_index("c")
    seq_base = wid * SEQ_PW
    idx = (idx0, idx1)
    rows = (rows0, rows1)
    gsem = (gs0, gs1)
    osem = (os0, os1)

    def start_gathers(s):
        b = s % 2
        seq0 = seq_base + s * SEQ_PC
        pltpu.sync_copy(x_hbm.at[pl.ds(seq0, SEQ_PC)], xbuf)
        zeros = jnp.zeros((16,), jnp.int32)
        for i in range(SEQ_PC):
            idx[b][pl.ds(i * HP + 40, 16)] = zeros
            for col in (0, 16, 32, HIST - 16):
                idx[b][pl.ds(i * HP + col, 16)] = xbuf[i, pl.ds(col, 16)]
        return [
            pltpu.async_copy(
                table_hbm.at[idx[b].at[pl.ds(g * 2 * HP, 2 * HP)]],
                rows[b].at[pl.ds(g * 2 * HP, 2 * HP)], gsem[b])
            for g in range(SEQ_PC // 2)
        ]

    gathers = [None] * STEPS
    writes = [None] * STEPS
    gathers[0] = start_gathers(0)
    for s in range(STEPS):
        b = s % 2
        if s + 1 < STEPS:
            if s >= 1:
                writes[s - 1].wait()
            gathers[s + 1] = start_gathers(s + 1)
        for g in gathers[s]:
            g.wait()

        def row(h, c):
            for i in range(SEQ_PC):
                for k in range(DIM // 16):
                    sl = pl.ds(k * 16, 16)
                    rows[b][i * HP + h, sl] = rows[b][i * HP + h, sl] * SCALE
            return c

        lax.fori_loop(0, HIST, row, 0)
        out0 = (seq_base + s * SEQ_PC) * HP
        writes[s] = pltpu.async_copy(
            rows[b], out_hbm.at[pl.ds(out0, SEQ_PC * HP)], osem[b])
    writes[STEPS - 2].wait()
    writes[STEPS - 1].wait()


def kernel(x, table):
    tbl128 = jnp.pad(table, ((0, 0), (0, DIM)))
    xp = jnp.pad(x, ((0, 0), (0, DP - HIST)))
    out = _emb_lookup(xp, tbl128)
    return out.reshape(BATCH, HP, DP)[:, :HIST, :DIM]


# speedup vs baseline: 6.3069x; 6.3064x over previous
import functools

import jax
import jax.numpy as jnp
from jax import lax
from jax.experimental import pallas as pl
from jax.experimental.pallas import tpu as pltpu
from jax.experimental.pallas import tpu_sc as plsc

VOCAB = 100000
DIM = 64
BATCH = 4096
HIST = 50

NC, NS = 2, 16
NW = NC * NS
SEQ_PW = BATCH // NW
SEQ_PC = 8
CHUNK = SEQ_PC * HIST
STEPS = SEQ_PW // SEQ_PC
SCALE = 8.0
HP = 56
DP = 2 * DIM

_mesh = plsc.VectorSubcoreMesh(core_axis_name="c", subcore_axis_name="s")


@functools.partial(
    pl.kernel,
    out_type=jax.ShapeDtypeStruct((BATCH, HP, DP), jnp.float32),
    mesh=_mesh,
    scratch_types=[
        pltpu.VMEM((SEQ_PC, DP), jnp.int32),
        pltpu.VMEM((SEQ_PC * HP + 16,), jnp.int32),
        pltpu.VMEM((SEQ_PC * HP + 16,), jnp.int32),
        pltpu.VMEM((SEQ_PC, HP, DP), jnp.float32),
        pltpu.VMEM((SEQ_PC, HP, DP), jnp.float32),
        pltpu.SemaphoreType.DMA,
        pltpu.SemaphoreType.DMA,
        pltpu.SemaphoreType.DMA,
        pltpu.SemaphoreType.DMA,
    ],
)
def _emb_lookup(x_hbm, table_hbm, out_hbm, xbuf, idx0, idx1, rows0, rows1,
                gs0, gs1, os0, os1):
    wid = lax.axis_index("s") * NC + lax.axis_index("c")
    seq_base = wid * SEQ_PW
    idx = (idx0, idx1)
    rows = (rows0, rows1)
    gsem = (gs0, gs1)
    osem = (os0, os1)

    def start_gathers(s):
        b = s % 2
        seq0 = seq_base + s * SEQ_PC
        pltpu.sync_copy(x_hbm.at[pl.ds(seq0, SEQ_PC)], xbuf)
        for i in range(SEQ_PC):
            for col in (0, 16, 32, HIST - 16):
                idx[b][pl.ds(i * HP + col, 16)] = xbuf[i, pl.ds(col, 16)]
        return [
            pltpu.async_copy(
                table_hbm.at[idx[b].at[pl.ds(i * HP, HIST)]],
                rows[b].at[i, pl.ds(0, HIST)], gsem[b])
            for i in range(SEQ_PC)
        ]

    gathers = [None] * STEPS
    writes = [None] * STEPS
    gathers[0] = start_gathers(0)
    for s in range(STEPS):
        b = s % 2
        if s + 1 < STEPS:
            if s >= 1:
                writes[s - 1].wait()
            gathers[s + 1] = start_gathers(s + 1)
        for g in gathers[s]:
            g.wait()

        def row(h, c):
            for i in range(SEQ_PC):
                for k in range(DIM // 16):
                    sl = pl.ds(k * 16, 16)
                    rows[b][i, h, sl] = rows[b][i, h, sl] * SCALE
            return c

        lax.fori_loop(0, HIST, row, 0)
        seq0 = seq_base + s * SEQ_PC
        writes[s] = pltpu.async_copy(
            rows[b], out_hbm.at[pl.ds(seq0, SEQ_PC)], osem[b])
    writes[STEPS - 2].wait()
    writes[STEPS - 1].wait()


def kernel(x, table):
    tbl128 = jnp.pad(table, ((0, 0), (0, DIM)))
    xp = jnp.pad(x, ((0, 0), (0, DP - HIST)))
    out = _emb_lookup(xp, tbl128)
    return out[:, :HIST, :DIM]
